# Initial kernel scaffold; baseline (speedup 1.0000x reference)
import jax
import jax.numpy as jnp
from jax.experimental import pallas as pl


def kernel(x_user, x_item, edge_index_user_rates_item, edge_index_item_rated_by_user, l1_ui_Wrel, l1_ui_Wroot, l1_ui_b, l1_iu_Wrel, l1_iu_Wroot, l1_iu_b, l2_ui_Wrel, l2_ui_Wroot, l2_ui_b, l2_iu_Wrel, l2_iu_Wroot, l2_iu_b, lin_user_W, lin_user_b, lin_item_W, lin_item_b):
    raise NotImplementedError("write your pallas kernel here")


# speedup vs baseline: 3.3960x; 3.3960x over previous
import functools

import jax
import jax.numpy as jnp
from jax import lax
from jax.experimental import pallas as pl
from jax.experimental.pallas import tpu as pltpu
from jax.experimental.pallas import tpu_sc as plsc

N = 10000
D_IN = 128
D_H = 256
D_OUT = 128
E = 320000
NT = 16
NC = 2
CH = 128
CHUNKS = E // CH
FULL = CHUNKS // NT
TAIL = CHUNKS - FULL * NT
OWN = 640
LAST = N - OWN * (NT - 1)
BR = 1000

@functools.lru_cache(maxsize=None)
def _mesh():
    return plsc.VectorSubcoreMesh(core_axis_name="c", subcore_axis_name="s",
                                  num_cores=NC, num_subcores=NT)


def _sc_layer1(x_hbm, src_hbm, dst_hbm, ones_hbm, zeros_hbm,
               sum_hbm, cnt_hbm,
               acc, idx_s, idx_d, rows, ones_v, sem):
    c = lax.axis_index("c")
    s = lax.axis_index("s")
    r0 = s * OWN
    base_e = c * E

    pltpu.sync_copy(ones_hbm, ones_v)

    for p in range(2):
        @pl.when(s < NT - 1)
        def _():
            pltpu.sync_copy(zeros_hbm.at[pl.ds(r0, OWN), :],
                            acc.at[pl.ds(r0, OWN), :])

        @pl.when(s == NT - 1)
        def _():
            pltpu.sync_copy(zeros_hbm.at[pl.ds(r0, LAST), :],
                            acc.at[pl.ds(r0, LAST), :])

        plsc.subcore_barrier()

        if p == 0:
            def chunk(off):
                pltpu.sync_copy(src_hbm.at[pl.ds(off, CH)], idx_s)
                pltpu.sync_copy(dst_hbm.at[pl.ds(off, CH)], idx_d)
                pltpu.async_copy(x_hbm.at[idx_s], rows, sem).wait()
                pltpu.sync_copy(rows, acc.at[idx_d], add=True)
        else:
            def chunk(off):
                pltpu.sync_copy(dst_hbm.at[pl.ds(off, CH)], idx_d)
                pltpu.sync_copy(ones_v, acc.at[idx_d], add=True)

        def body(j, carry):
            chunk(base_e + (s + j * NT) * CH)
            return carry

        lax.fori_loop(0, FULL, body, 0)

        @pl.when(s < TAIL)
        def _():
            chunk(base_e + (FULL * NT + s) * CH)

        plsc.subcore_barrier()

        out_hbm = sum_hbm if p == 0 else cnt_hbm

        @pl.when(s < NT - 1)
        def _():
            pltpu.sync_copy(acc.at[pl.ds(r0, OWN), :],
                            out_hbm.at[pl.ds(c * N + r0, OWN), :])

        @pl.when(s == NT - 1)
        def _():
            pltpu.sync_copy(acc.at[pl.ds(r0, LAST), :],
                            out_hbm.at[pl.ds(c * N + r0, LAST), :])

        if p == 0:
            plsc.subcore_barrier()


@functools.lru_cache(maxsize=None)
def _layer1_call():
    return pl.kernel(
        _sc_layer1,
        out_type=[jax.ShapeDtypeStruct((NC * N, D_IN), jnp.float32),
                  jax.ShapeDtypeStruct((NC * N, 128), jnp.float32)],
        mesh=_mesh(),
        scratch_types=[
            pltpu.VMEM_SHARED((N, D_IN), jnp.float32),
            pltpu.VMEM((CH,), jnp.int32),
            pltpu.VMEM((CH,), jnp.int32),
            pltpu.VMEM((CH, D_IN), jnp.float32),
            pltpu.VMEM((CH, 128), jnp.float32),
            pltpu.SemaphoreType.DMA,
        ],
    )


def _sc_layer2(h_hbm, src_hbm, dst_hbm, zeros_hbm,
               sum_hbm,
               acc, idx_s, idx_d, rows, sem):
    c = lax.axis_index("c")
    s = lax.axis_index("s")
    r0 = s * OWN

    for p in range(2):
        @pl.when(s < NT - 1)
        def _():
            pltpu.sync_copy(zeros_hbm.at[pl.ds(r0, OWN), :],
                            acc.at[pl.ds(r0, OWN), :])

        @pl.when(s == NT - 1)
        def _():
            pltpu.sync_copy(zeros_hbm.at[pl.ds(r0, LAST), :],
                            acc.at[pl.ds(r0, LAST), :])

        plsc.subcore_barrier()

        base_e = (2 * c + p) * E
        base_d = c * E

        def chunk(ch_id):
            pltpu.sync_copy(src_hbm.at[pl.ds(base_e + ch_id * CH, CH)], idx_s)
            pltpu.sync_copy(dst_hbm.at[pl.ds(base_d + ch_id * CH, CH)], idx_d)
            pltpu.async_copy(h_hbm.at[idx_s], rows, sem).wait()
            pltpu.sync_copy(rows, acc.at[idx_d], add=True)

        def body(j, carry):
            chunk(s + j * NT)
            return carry

        lax.fori_loop(0, FULL, body, 0)

        @pl.when(s < TAIL)
        def _():
            chunk(FULL * NT + s)

        plsc.subcore_barrier()

        @pl.when(s < NT - 1)
        def _():
            pltpu.sync_copy(
                acc.at[pl.ds(r0, OWN), :],
                sum_hbm.at[pl.ds(c * N + r0, OWN), pl.ds(p * 128, 128)])

        @pl.when(s == NT - 1)
        def _():
            pltpu.sync_copy(
                acc.at[pl.ds(r0, LAST), :],
                sum_hbm.at[pl.ds(c * N + r0, LAST), pl.ds(p * 128, 128)])

        plsc.subcore_barrier()


@functools.lru_cache(maxsize=None)
def _layer2_call():
    return pl.kernel(
        _sc_layer2,
        out_type=[jax.ShapeDtypeStruct((NC * N, D_H), jnp.float32)],
        mesh=_mesh(),
        scratch_types=[
            pltpu.VMEM_SHARED((N, 128), jnp.float32),
            pltpu.VMEM((CH,), jnp.int32),
            pltpu.VMEM((CH,), jnp.int32),
            pltpu.VMEM((CH, 128), jnp.float32),
            pltpu.SemaphoreType.DMA,
        ],
    )


def _tc_dense1(sum_ref, cnt_ref, xr_ref, wrel_ref, wroot_ref, b_ref, out_ref):
    cnt = jnp.maximum(cnt_ref[:, 0:1], 1.0)
    agg = sum_ref[...] / cnt
    h = jnp.dot(agg, wrel_ref[0], preferred_element_type=jnp.float32)
    h = h + jnp.dot(xr_ref[...], wroot_ref[0], preferred_element_type=jnp.float32)
    h = h + b_ref[0]
    h = jnp.maximum(h, 0.0)
    out_ref[0] = h[:, :128]
    out_ref[1] = h[:, 128:]


_dense1_call = pl.pallas_call(
    _tc_dense1,
    grid=(2, N // BR),
    in_specs=[
        pl.BlockSpec((BR, D_IN), lambda t, r: (t * (N // BR) + r, 0)),
        pl.BlockSpec((BR, 128), lambda t, r: (t * (N // BR) + r, 0)),
        pl.BlockSpec((BR, D_IN), lambda t, r: (t * (N // BR) + r, 0)),
        pl.BlockSpec((1, D_IN, D_H), lambda t, r: (t, 0, 0)),
        pl.BlockSpec((1, D_IN, D_H), lambda t, r: (t, 0, 0)),
        pl.BlockSpec((1, 1, D_H), lambda t, r: (t, 0, 0)),
    ],
    out_specs=pl.BlockSpec((2, BR, 128), lambda t, r: (1 - t, r, 0)),
    out_shape=jax.ShapeDtypeStruct((4, N, 128), jnp.float32),
)


def _tc_dense2(sum_ref, cnt_ref, h4_ref, wrel_ref, wroot_ref, b_ref,
               lw_ref, lb_ref, out_ref):
    cnt = jnp.maximum(cnt_ref[:, 0:1], 1.0)
    agg = sum_ref[...] / cnt
    o = jnp.dot(agg, wrel_ref[0], preferred_element_type=jnp.float32)
    o = o + jnp.dot(h4_ref[0], wroot_ref[0, :128, :],
                    preferred_element_type=jnp.float32)
    o = o + jnp.dot(h4_ref[1], wroot_ref[0, 128:, :],
                    preferred_element_type=jnp.float32)
    o = o + b_ref[0]
    out_ref[0] = jnp.dot(o, lw_ref[0], preferred_element_type=jnp.float32) \
        + lb_ref[0]


_dense2_call = pl.pallas_call(
    _tc_dense2,
    grid=(2, N // BR),
    in_specs=[
        pl.BlockSpec((BR, D_H), lambda t, r: (t * (N // BR) + r, 0)),
        pl.BlockSpec((BR, 128), lambda t, r: (t * (N // BR) + r, 0)),
        pl.BlockSpec((2, BR, 128), lambda t, r: (1 - t, r, 0)),
        pl.BlockSpec((1, D_H, D_OUT), lambda t, r: (t, 0, 0)),
        pl.BlockSpec((1, D_H, D_OUT), lambda t, r: (t, 0, 0)),
        pl.BlockSpec((1, 1, D_OUT), lambda t, r: (t, 0, 0)),
        pl.BlockSpec((1, D_OUT, D_OUT), lambda t, r: (t, 0, 0)),
        pl.BlockSpec((1, 1, D_OUT), lambda t, r: (t, 0, 0)),
    ],
    out_specs=pl.BlockSpec((1, BR, D_OUT), lambda t, r: (t, r, 0)),
    out_shape=jax.ShapeDtypeStruct((2, N, D_OUT), jnp.float32),
)


def kernel(x_user, x_item, edge_index_user_rates_item, edge_index_item_rated_by_user,
           l1_ui_Wrel, l1_ui_Wroot, l1_ui_b, l1_iu_Wrel, l1_iu_Wroot, l1_iu_b,
           l2_ui_Wrel, l2_ui_Wroot, l2_ui_b, l2_iu_Wrel, l2_iu_Wroot, l2_iu_b,
           lin_user_W, lin_user_b, lin_item_W, lin_item_b):
    src_ui = edge_index_user_rates_item[0].astype(jnp.int32)
    dst_ui = edge_index_user_rates_item[1].astype(jnp.int32)
    src_iu = edge_index_item_rated_by_user[0].astype(jnp.int32)
    dst_iu = edge_index_item_rated_by_user[1].astype(jnp.int32)

    x_all = jnp.concatenate([x_user, x_item], axis=0)
    src1 = jnp.concatenate([src_ui, src_iu + N])
    dst1 = jnp.concatenate([dst_ui, dst_iu])
    src2 = jnp.concatenate([src_ui, src_ui + N,
                            src_iu + 2 * N, src_iu + 3 * N])
    ones128 = jnp.ones((CH, 128), jnp.float32)
    zeros = jnp.zeros((N, D_IN), jnp.float32)

    sum1, cnt = _layer1_call()(x_all, src1, dst1, ones128, zeros)

    xr = jnp.concatenate([x_item, x_user], axis=0)
    w1rel = jnp.stack([l1_ui_Wrel, l1_iu_Wrel])
    w1root = jnp.stack([l1_ui_Wroot, l1_iu_Wroot])
    b1 = jnp.stack([l1_ui_b, l1_iu_b])[:, None, :]
    h4 = _dense1_call(sum1, cnt, xr, w1rel, w1root, b1)

    h_tab = h4.reshape(4 * N, 128)
    (sum2,) = _layer2_call()(h_tab, src2, dst1, zeros)

    w2rel = jnp.stack([l2_ui_Wrel, l2_iu_Wrel])
    w2root = jnp.stack([l2_ui_Wroot, l2_iu_Wroot])
    b2 = jnp.stack([l2_ui_b, l2_iu_b])[:, None, :]
    lw = jnp.stack([lin_item_W, lin_user_W])
    lb = jnp.stack([lin_item_b, lin_user_b])[:, None, :]
    out = _dense2_call(sum2, cnt, h4, w2rel, w2root, b2, lw, lb)
    return (out[1], out[0])


# speedup vs baseline: 5.3909x; 1.5874x over previous
import functools

import jax
import jax.numpy as jnp
from jax import lax
from jax.experimental import pallas as pl
from jax.experimental.pallas import tpu as pltpu
from jax.experimental.pallas import tpu_sc as plsc

N = 10000
D_IN = 128
D_H = 256
D_OUT = 128
E = 320000
NT = 16
NC = 2
CH = 128
CHUNKS = E // CH
FULL = CHUNKS // NT
HFULL = FULL // 2
TAIL = CHUNKS - FULL * NT
OWN = 640
LAST = N - OWN * (NT - 1)
BR = 1000


@functools.lru_cache(maxsize=None)
def _mesh():
    return plsc.VectorSubcoreMesh(core_axis_name="c", subcore_axis_name="s",
                                  num_cores=NC, num_subcores=NT)


def _sc_layer1(x_hbm, src_hbm, dst_hbm, ones_hbm, zeros_hbm,
               sum_hbm, cnt_hbm,
               acc, isa, ida, isb, idb, ra, rb, ones_v,
               sga, sgb, ssa, ssb, sem):
    c = lax.axis_index("c")
    s = lax.axis_index("s")
    r0 = s * OWN
    base_e = c * E

    def eoff(k):
        return base_e + (s + k * NT) * CH

    pltpu.sync_copy(ones_hbm, ones_v)

    for p in range(2):
        @pl.when(s < NT - 1)
        def _():
            pltpu.sync_copy(zeros_hbm.at[pl.ds(r0, OWN), :],
                            acc.at[pl.ds(r0, OWN), :])

        @pl.when(s == NT - 1)
        def _():
            pltpu.sync_copy(zeros_hbm.at[pl.ds(r0, LAST), :],
                            acc.at[pl.ds(r0, LAST), :])

        plsc.subcore_barrier()

        if p == 0:
            pltpu.sync_copy(src_hbm.at[pl.ds(eoff(0), CH)], isa)
            pltpu.sync_copy(dst_hbm.at[pl.ds(eoff(0), CH)], ida)
            pltpu.async_copy(x_hbm.at[isa], ra, sga)

            def body(j, carry):
                @pl.when(j > 0)
                def _():
                    pltpu.make_async_copy(rb, acc.at[idb], ssb).wait()

                pltpu.sync_copy(src_hbm.at[pl.ds(eoff(2 * j + 1), CH)], isb)
                pltpu.sync_copy(dst_hbm.at[pl.ds(eoff(2 * j + 1), CH)], idb)
                pltpu.make_async_copy(x_hbm.at[isa], ra, sga).wait()
                pltpu.async_copy(x_hbm.at[isb], rb, sgb)
                pltpu.async_copy(ra, acc.at[ida], ssa, add=True)
                pltpu.make_async_copy(ra, acc.at[ida], ssa).wait()

                @pl.when(j < HFULL - 1)
                def _():
                    pltpu.sync_copy(
                        src_hbm.at[pl.ds(eoff(2 * j + 2), CH)], isa)
                    pltpu.sync_copy(
                        dst_hbm.at[pl.ds(eoff(2 * j + 2), CH)], ida)
                    pltpu.async_copy(x_hbm.at[isa], ra, sga)

                pltpu.make_async_copy(x_hbm.at[isb], rb, sgb).wait()
                pltpu.async_copy(rb, acc.at[idb], ssb, add=True)
                return carry

            lax.fori_loop(0, HFULL, body, 0)
            pltpu.make_async_copy(rb, acc.at[idb], ssb).wait()

            @pl.when(s < TAIL)
            def _():
                off = base_e + (FULL * NT + s) * CH
                pltpu.sync_copy(src_hbm.at[pl.ds(off, CH)], isa)
                pltpu.sync_copy(dst_hbm.at[pl.ds(off, CH)], ida)
                pltpu.async_copy(x_hbm.at[isa], ra, sem).wait()
                pltpu.sync_copy(ra, acc.at[ida], add=True)
        else:
            pltpu.sync_copy(dst_hbm.at[pl.ds(eoff(0), CH)], ida)

            def body(j, carry):
                pltpu.async_copy(ones_v, acc.at[ida], ssa, add=True)

                @pl.when(j > 0)
                def _():
                    pltpu.make_async_copy(ones_v, acc.at[idb], ssb).wait()

                pltpu.sync_copy(dst_hbm.at[pl.ds(eoff(2 * j + 1), CH)], idb)
                pltpu.async_copy(ones_v, acc.at[idb], ssb, add=True)
                pltpu.make_async_copy(ones_v, acc.at[ida], ssa).wait()

                @pl.when(j < HFULL - 1)
                def _():
                    pltpu.sync_copy(
                        dst_hbm.at[pl.ds(eoff(2 * j + 2), CH)], ida)

                return carry

            lax.fori_loop(0, HFULL, body, 0)
            pltpu.make_async_copy(ones_v, acc.at[idb], ssb).wait()

            @pl.when(s < TAIL)
            def _():
                off = base_e + (FULL * NT + s) * CH
                pltpu.sync_copy(dst_hbm.at[pl.ds(off, CH)], ida)
                pltpu.sync_copy(ones_v, acc.at[ida], add=True)

        plsc.subcore_barrier()

        out_hbm = sum_hbm if p == 0 else cnt_hbm

        @pl.when(s < NT - 1)
        def _():
            pltpu.sync_copy(acc.at[pl.ds(r0, OWN), :],
                            out_hbm.at[pl.ds(c * N + r0, OWN), :])

        @pl.when(s == NT - 1)
        def _():
            pltpu.sync_copy(acc.at[pl.ds(r0, LAST), :],
                            out_hbm.at[pl.ds(c * N + r0, LAST), :])

        if p == 0:
            plsc.subcore_barrier()


@functools.lru_cache(maxsize=None)
def _layer1_call():
    return pl.kernel(
        _sc_layer1,
        out_type=[jax.ShapeDtypeStruct((NC * N, D_IN), jnp.float32),
                  jax.ShapeDtypeStruct((NC * N, 128), jnp.float32)],
        mesh=_mesh(),
        scratch_types=[
            pltpu.VMEM_SHARED((N, D_IN), jnp.float32),
            pltpu.VMEM((CH,), jnp.int32),
            pltpu.VMEM((CH,), jnp.int32),
            pltpu.VMEM((CH,), jnp.int32),
            pltpu.VMEM((CH,), jnp.int32),
            pltpu.VMEM((CH, D_IN), jnp.float32),
            pltpu.VMEM((CH, D_IN), jnp.float32),
            pltpu.VMEM((CH, 128), jnp.float32),
            pltpu.SemaphoreType.DMA,
            pltpu.SemaphoreType.DMA,
            pltpu.SemaphoreType.DMA,
            pltpu.SemaphoreType.DMA,
            pltpu.SemaphoreType.DMA,
        ],
    )


def _sc_layer2(h_hbm, src_hbm, dst_hbm, zeros_hbm,
               sum_hbm,
               acc, isa, ida, isb, idb, ra, rb,
               sga, sgb, ssa, ssb, sem):
    c = lax.axis_index("c")
    s = lax.axis_index("s")
    r0 = s * OWN

    for p in range(2):
        base_e = (2 * c + p) * E
        base_d = c * E

        def eoff(k):
            return base_e + (s + k * NT) * CH

        def doff(k):
            return base_d + (s + k * NT) * CH

        @pl.when(s < NT - 1)
        def _():
            pltpu.sync_copy(zeros_hbm.at[pl.ds(r0, OWN), :],
                            acc.at[pl.ds(r0, OWN), :])

        @pl.when(s == NT - 1)
        def _():
            pltpu.sync_copy(zeros_hbm.at[pl.ds(r0, LAST), :],
                            acc.at[pl.ds(r0, LAST), :])

        plsc.subcore_barrier()

        pltpu.sync_copy(src_hbm.at[pl.ds(eoff(0), CH)], isa)
        pltpu.sync_copy(dst_hbm.at[pl.ds(doff(0), CH)], ida)
        pltpu.async_copy(h_hbm.at[isa], ra, sga)

        def body(j, carry):
            @pl.when(j > 0)
            def _():
                pltpu.make_async_copy(rb, acc.at[idb], ssb).wait()

            pltpu.sync_copy(src_hbm.at[pl.ds(eoff(2 * j + 1), CH)], isb)
            pltpu.sync_copy(dst_hbm.at[pl.ds(doff(2 * j + 1), CH)], idb)
            pltpu.make_async_copy(h_hbm.at[isa], ra, sga).wait()
            pltpu.async_copy(h_hbm.at[isb], rb, sgb)
            pltpu.async_copy(ra, acc.at[ida], ssa, add=True)
            pltpu.make_async_copy(ra, acc.at[ida], ssa).wait()

            @pl.when(j < HFULL - 1)
            def _():
                pltpu.sync_copy(src_hbm.at[pl.ds(eoff(2 * j + 2), CH)], isa)
                pltpu.sync_copy(dst_hbm.at[pl.ds(doff(2 * j + 2), CH)], ida)
                pltpu.async_copy(h_hbm.at[isa], ra, sga)

            pltpu.make_async_copy(h_hbm.at[isb], rb, sgb).wait()
            pltpu.async_copy(rb, acc.at[idb], ssb, add=True)
            return carry

        lax.fori_loop(0, HFULL, body, 0)
        pltpu.make_async_copy(rb, acc.at[idb], ssb).wait()

        @pl.when(s < TAIL)
        def _():
            off_e = base_e + (FULL * NT + s) * CH
            off_d = base_d + (FULL * NT + s) * CH
            pltpu.sync_copy(src_hbm.at[pl.ds(off_e, CH)], isa)
            pltpu.sync_copy(dst_hbm.at[pl.ds(off_d, CH)], ida)
            pltpu.async_copy(h_hbm.at[isa], ra, sem).wait()
            pltpu.sync_copy(ra, acc.at[ida], add=True)

        plsc.subcore_barrier()

        @pl.when(s < NT - 1)
        def _():
            pltpu.sync_copy(
                acc.at[pl.ds(r0, OWN), :],
                sum_hbm.at[pl.ds(c * N + r0, OWN), pl.ds(p * 128, 128)])

        @pl.when(s == NT - 1)
        def _():
            pltpu.sync_copy(
                acc.at[pl.ds(r0, LAST), :],
                sum_hbm.at[pl.ds(c * N + r0, LAST), pl.ds(p * 128, 128)])

        plsc.subcore_barrier()


@functools.lru_cache(maxsize=None)
def _layer2_call():
    return pl.kernel(
        _sc_layer2,
        out_type=[jax.ShapeDtypeStruct((NC * N, D_H), jnp.float32)],
        mesh=_mesh(),
        scratch_types=[
            pltpu.VMEM_SHARED((N, 128), jnp.float32),
            pltpu.VMEM((CH,), jnp.int32),
            pltpu.VMEM((CH,), jnp.int32),
            pltpu.VMEM((CH,), jnp.int32),
            pltpu.VMEM((CH,), jnp.int32),
            pltpu.VMEM((CH, 128), jnp.float32),
            pltpu.VMEM((CH, 128), jnp.float32),
            pltpu.SemaphoreType.DMA,
            pltpu.SemaphoreType.DMA,
            pltpu.SemaphoreType.DMA,
            pltpu.SemaphoreType.DMA,
            pltpu.SemaphoreType.DMA,
        ],
    )


def _tc_dense1(sum_ref, cnt_ref, xr_ref, wrel_ref, wroot_ref, b_ref, out_ref):
    cnt = jnp.maximum(cnt_ref[:, 0:1], 1.0)
    agg = sum_ref[...] / cnt
    h = jnp.dot(agg, wrel_ref[0], preferred_element_type=jnp.float32)
    h = h + jnp.dot(xr_ref[...], wroot_ref[0], preferred_element_type=jnp.float32)
    h = h + b_ref[0]
    h = jnp.maximum(h, 0.0)
    out_ref[0] = h[:, :128]
    out_ref[1] = h[:, 128:]


_dense1_call = pl.pallas_call(
    _tc_dense1,
    grid=(2, N // BR),
    in_specs=[
        pl.BlockSpec((BR, D_IN), lambda t, r: (t * (N // BR) + r, 0)),
        pl.BlockSpec((BR, 128), lambda t, r: (t * (N // BR) + r, 0)),
        pl.BlockSpec((BR, D_IN), lambda t, r: (t * (N // BR) + r, 0)),
        pl.BlockSpec((1, D_IN, D_H), lambda t, r: (t, 0, 0)),
        pl.BlockSpec((1, D_IN, D_H), lambda t, r: (t, 0, 0)),
        pl.BlockSpec((1, 1, D_H), lambda t, r: (t, 0, 0)),
    ],
    out_specs=pl.BlockSpec((2, BR, 128), lambda t, r: (1 - t, r, 0)),
    out_shape=jax.ShapeDtypeStruct((4, N, 128), jnp.float32),
)


def _tc_dense2(sum_ref, cnt_ref, h4_ref, wrel_ref, wroot_ref, b_ref,
               lw_ref, lb_ref, out_ref):
    cnt = jnp.maximum(cnt_ref[:, 0:1], 1.0)
    agg = sum_ref[...] / cnt
    o = jnp.dot(agg, wrel_ref[0], preferred_element_type=jnp.float32)
    o = o + jnp.dot(h4_ref[0], wroot_ref[0, :128, :],
                    preferred_element_type=jnp.float32)
    o = o + jnp.dot(h4_ref[1], wroot_ref[0, 128:, :],
                    preferred_element_type=jnp.float32)
    o = o + b_ref[0]
    out_ref[0] = jnp.dot(o, lw_ref[0], preferred_element_type=jnp.float32) \
        + lb_ref[0]


_dense2_call = pl.pallas_call(
    _tc_dense2,
    grid=(2, N // BR),
    in_specs=[
        pl.BlockSpec((BR, D_H), lambda t, r: (t * (N // BR) + r, 0)),
        pl.BlockSpec((BR, 128), lambda t, r: (t * (N // BR) + r, 0)),
        pl.BlockSpec((2, BR, 128), lambda t, r: (1 - t, r, 0)),
        pl.BlockSpec((1, D_H, D_OUT), lambda t, r: (t, 0, 0)),
        pl.BlockSpec((1, D_H, D_OUT), lambda t, r: (t, 0, 0)),
        pl.BlockSpec((1, 1, D_OUT), lambda t, r: (t, 0, 0)),
        pl.BlockSpec((1, D_OUT, D_OUT), lambda t, r: (t, 0, 0)),
        pl.BlockSpec((1, 1, D_OUT), lambda t, r: (t, 0, 0)),
    ],
    out_specs=pl.BlockSpec((1, BR, D_OUT), lambda t, r: (t, r, 0)),
    out_shape=jax.ShapeDtypeStruct((2, N, D_OUT), jnp.float32),
)


def kernel(x_user, x_item, edge_index_user_rates_item, edge_index_item_rated_by_user,
           l1_ui_Wrel, l1_ui_Wroot, l1_ui_b, l1_iu_Wrel, l1_iu_Wroot, l1_iu_b,
           l2_ui_Wrel, l2_ui_Wroot, l2_ui_b, l2_iu_Wrel, l2_iu_Wroot, l2_iu_b,
           lin_user_W, lin_user_b, lin_item_W, lin_item_b):
    src_ui = edge_index_user_rates_item[0].astype(jnp.int32)
    dst_ui = edge_index_user_rates_item[1].astype(jnp.int32)
    src_iu = edge_index_item_rated_by_user[0].astype(jnp.int32)
    dst_iu = edge_index_item_rated_by_user[1].astype(jnp.int32)

    x_all = jnp.concatenate([x_user, x_item], axis=0)
    src1 = jnp.concatenate([src_ui, src_iu + N])
    dst1 = jnp.concatenate([dst_ui, dst_iu])
    src2 = jnp.concatenate([src_ui, src_ui + N,
                            src_iu + 2 * N, src_iu + 3 * N])
    ones128 = jnp.ones((CH, 128), jnp.float32)
    zeros = jnp.zeros((N, D_IN), jnp.float32)

    sum1, cnt = _layer1_call()(x_all, src1, dst1, ones128, zeros)

    xr = jnp.concatenate([x_item, x_user], axis=0)
    w1rel = jnp.stack([l1_ui_Wrel, l1_iu_Wrel])
    w1root = jnp.stack([l1_ui_Wroot, l1_iu_Wroot])
    b1 = jnp.stack([l1_ui_b, l1_iu_b])[:, None, :]
    h4 = _dense1_call(sum1, cnt, xr, w1rel, w1root, b1)

    h_tab = h4.reshape(4 * N, 128)
    (sum2,) = _layer2_call()(h_tab, src2, dst1, zeros)

    w2rel = jnp.stack([l2_ui_Wrel, l2_iu_Wrel])
    w2root = jnp.stack([l2_ui_Wroot, l2_iu_Wroot])
    b2 = jnp.stack([l2_ui_b, l2_iu_b])[:, None, :]
    lw = jnp.stack([lin_item_W, lin_user_W])
    lb = jnp.stack([lin_item_b, lin_user_b])[:, None, :]
    out = _dense2_call(sum2, cnt, h4, w2rel, w2root, b2, lw, lb)
    return (out[1], out[0])


# speedup vs baseline: 6.0089x; 1.1146x over previous
import functools

import jax
import jax.numpy as jnp
from jax import lax
from jax.experimental import pallas as pl
from jax.experimental.pallas import tpu as pltpu
from jax.experimental.pallas import tpu_sc as plsc

N = 10000
D_IN = 128
D_H = 256
D_OUT = 128
E = 320000
NT = 16
NC = 2
CH = 128
CHUNKS = E // CH
FULL = CHUNKS // NT
HFULL = FULL // 2
TAIL = CHUNKS - FULL * NT
OWN = 640
LAST = N - OWN * (NT - 1)
BR = 1000


@functools.lru_cache(maxsize=None)
def _mesh():
    return plsc.VectorSubcoreMesh(core_axis_name="c", subcore_axis_name="s",
                                  num_cores=NC, num_subcores=NT)


def _sc_layer1(x_hbm, src_hbm, dst_hbm, ones_hbm, zeros_hbm,
               sum_hbm, cnt_hbm,
               acc, isa, ida, isb, idb, ra, rb, ones_v,
               sga, sgb, ssa, ssb, sia, sib, sem):
    c = lax.axis_index("c")
    s = lax.axis_index("s")
    r0 = s * OWN
    base_e = c * E

    def eoff(k):
        return base_e + (s + k * NT) * CH

    pltpu.sync_copy(ones_hbm, ones_v)

    for p in range(2):
        @pl.when(s < NT - 1)
        def _():
            pltpu.sync_copy(zeros_hbm.at[pl.ds(r0, OWN), :],
                            acc.at[pl.ds(r0, OWN), :])

        @pl.when(s == NT - 1)
        def _():
            pltpu.sync_copy(zeros_hbm.at[pl.ds(r0, LAST), :],
                            acc.at[pl.ds(r0, LAST), :])

        plsc.subcore_barrier()

        if p == 0:
            pltpu.sync_copy(src_hbm.at[pl.ds(eoff(0), CH)], isa)
            pltpu.sync_copy(dst_hbm.at[pl.ds(eoff(0), CH)], ida)
            pltpu.async_copy(x_hbm.at[isa], ra, sga)

            def body(j, carry):
                @pl.when(j > 0)
                def _():
                    pltpu.make_async_copy(rb, acc.at[idb], ssb).wait()

                pltpu.async_copy(src_hbm.at[pl.ds(eoff(2 * j + 1), CH)],
                                 isb, sib)
                pltpu.async_copy(dst_hbm.at[pl.ds(eoff(2 * j + 1), CH)],
                                 idb, sib)
                pltpu.make_async_copy(x_hbm.at[isa], ra, sga).wait()
                pltpu.make_async_copy(
                    src_hbm.at[pl.ds(eoff(0), CH)], isb, sib).wait()
                pltpu.make_async_copy(
                    dst_hbm.at[pl.ds(eoff(0), CH)], idb, sib).wait()
                pltpu.async_copy(x_hbm.at[isb], rb, sgb)
                pltpu.async_copy(ra, acc.at[ida], ssa, add=True)
                pltpu.make_async_copy(ra, acc.at[ida], ssa).wait()

                @pl.when(j < HFULL - 1)
                def _():
                    pltpu.async_copy(
                        src_hbm.at[pl.ds(eoff(2 * j + 2), CH)], isa, sia)
                    pltpu.async_copy(
                        dst_hbm.at[pl.ds(eoff(2 * j + 2), CH)], ida, sia)

                pltpu.make_async_copy(x_hbm.at[isb], rb, sgb).wait()
                pltpu.async_copy(rb, acc.at[idb], ssb, add=True)

                @pl.when(j < HFULL - 1)
                def _():
                    pltpu.make_async_copy(
                        src_hbm.at[pl.ds(eoff(0), CH)], isa, sia).wait()
                    pltpu.make_async_copy(
                        dst_hbm.at[pl.ds(eoff(0), CH)], ida, sia).wait()
                    pltpu.async_copy(x_hbm.at[isa], ra, sga)

                return carry

            lax.fori_loop(0, HFULL, body, 0)
            pltpu.make_async_copy(rb, acc.at[idb], ssb).wait()

            @pl.when(s < TAIL)
            def _():
                off = base_e + (FULL * NT + s) * CH
                pltpu.sync_copy(src_hbm.at[pl.ds(off, CH)], isa)
                pltpu.sync_copy(dst_hbm.at[pl.ds(off, CH)], ida)
                pltpu.async_copy(x_hbm.at[isa], ra, sem).wait()
                pltpu.sync_copy(ra, acc.at[ida], add=True)
        else:
            pltpu.sync_copy(dst_hbm.at[pl.ds(eoff(0), CH)], ida)

            def body(j, carry):
                pltpu.async_copy(ones_v, acc.at[ida], ssa, add=True)

                @pl.when(j > 0)
                def _():
                    pltpu.make_async_copy(ones_v, acc.at[idb], ssb).wait()

                pltpu.sync_copy(dst_hbm.at[pl.ds(eoff(2 * j + 1), CH)], idb)
                pltpu.async_copy(ones_v, acc.at[idb], ssb, add=True)
                pltpu.make_async_copy(ones_v, acc.at[ida], ssa).wait()

                @pl.when(j < HFULL - 1)
                def _():
                    pltpu.sync_copy(
                        dst_hbm.at[pl.ds(eoff(2 * j + 2), CH)], ida)

                return carry

            lax.fori_loop(0, HFULL, body, 0)
            pltpu.make_async_copy(ones_v, acc.at[idb], ssb).wait()

            @pl.when(s < TAIL)
            def _():
                off = base_e + (FULL * NT + s) * CH
                pltpu.sync_copy(dst_hbm.at[pl.ds(off, CH)], ida)
                pltpu.sync_copy(ones_v, acc.at[ida], add=True)

        plsc.subcore_barrier()

        out_hbm = sum_hbm if p == 0 else cnt_hbm

        @pl.when(s < NT - 1)
        def _():
            pltpu.sync_copy(acc.at[pl.ds(r0, OWN), :],
                            out_hbm.at[pl.ds(c * N + r0, OWN), :])

        @pl.when(s == NT - 1)
        def _():
            pltpu.sync_copy(acc.at[pl.ds(r0, LAST), :],
                            out_hbm.at[pl.ds(c * N + r0, LAST), :])

        if p == 0:
            plsc.subcore_barrier()


@functools.lru_cache(maxsize=None)
def _layer1_call():
    return pl.kernel(
        _sc_layer1,
        out_type=[jax.ShapeDtypeStruct((NC * N, D_IN), jnp.float32),
                  jax.ShapeDtypeStruct((NC * N, 128), jnp.float32)],
        mesh=_mesh(),
        scratch_types=[
            pltpu.VMEM_SHARED((N, D_IN), jnp.float32),
            pltpu.VMEM((CH,), jnp.int32),
            pltpu.VMEM((CH,), jnp.int32),
            pltpu.VMEM((CH,), jnp.int32),
            pltpu.VMEM((CH,), jnp.int32),
            pltpu.VMEM((CH, D_IN), jnp.float32),
            pltpu.VMEM((CH, D_IN), jnp.float32),
            pltpu.VMEM((CH, 128), jnp.float32),
            pltpu.SemaphoreType.DMA,
            pltpu.SemaphoreType.DMA,
            pltpu.SemaphoreType.DMA,
            pltpu.SemaphoreType.DMA,
            pltpu.SemaphoreType.DMA,
            pltpu.SemaphoreType.DMA,
            pltpu.SemaphoreType.DMA,
        ],
    )


def _sc_layer2(h_hbm, src_hbm, dst_hbm, zeros_hbm,
               sum_hbm,
               acc, isa, ida, isb, idb, ra, rb,
               sga, sgb, ssa, ssb, sia, sib, sem):
    c = lax.axis_index("c")
    s = lax.axis_index("s")
    r0 = s * OWN

    for p in range(2):
        base_e = (2 * c + p) * E
        base_d = c * E

        def eoff(k):
            return base_e + (s + k * NT) * CH

        def doff(k):
            return base_d + (s + k * NT) * CH

        @pl.when(s < NT - 1)
        def _():
            pltpu.sync_copy(zeros_hbm.at[pl.ds(r0, OWN), :],
                            acc.at[pl.ds(r0, OWN), :])

        @pl.when(s == NT - 1)
        def _():
            pltpu.sync_copy(zeros_hbm.at[pl.ds(r0, LAST), :],
                            acc.at[pl.ds(r0, LAST), :])

        plsc.subcore_barrier()

        pltpu.sync_copy(src_hbm.at[pl.ds(eoff(0), CH)], isa)
        pltpu.sync_copy(dst_hbm.at[pl.ds(doff(0), CH)], ida)
        pltpu.async_copy(h_hbm.at[isa], ra, sga)

        def body(j, carry):
            @pl.when(j > 0)
            def _():
                pltpu.make_async_copy(rb, acc.at[idb], ssb).wait()

            pltpu.async_copy(src_hbm.at[pl.ds(eoff(2 * j + 1), CH)],
                             isb, sib)
            pltpu.async_copy(dst_hbm.at[pl.ds(doff(2 * j + 1), CH)],
                             idb, sib)
            pltpu.make_async_copy(h_hbm.at[isa], ra, sga).wait()
            pltpu.make_async_copy(
                src_hbm.at[pl.ds(eoff(0), CH)], isb, sib).wait()
            pltpu.make_async_copy(
                dst_hbm.at[pl.ds(doff(0), CH)], idb, sib).wait()
            pltpu.async_copy(h_hbm.at[isb], rb, sgb)
            pltpu.async_copy(ra, acc.at[ida], ssa, add=True)
            pltpu.make_async_copy(ra, acc.at[ida], ssa).wait()

            @pl.when(j < HFULL - 1)
            def _():
                pltpu.async_copy(
                    src_hbm.at[pl.ds(eoff(2 * j + 2), CH)], isa, sia)
                pltpu.async_copy(
                    dst_hbm.at[pl.ds(doff(2 * j + 2), CH)], ida, sia)

            pltpu.make_async_copy(h_hbm.at[isb], rb, sgb).wait()
            pltpu.async_copy(rb, acc.at[idb], ssb, add=True)

            @pl.when(j < HFULL - 1)
            def _():
                pltpu.make_async_copy(
                    src_hbm.at[pl.ds(eoff(0), CH)], isa, sia).wait()
                pltpu.make_async_copy(
                    dst_hbm.at[pl.ds(doff(0), CH)], ida, sia).wait()
                pltpu.async_copy(h_hbm.at[isa], ra, sga)

            return carry

        lax.fori_loop(0, HFULL, body, 0)
        pltpu.make_async_copy(rb, acc.at[idb], ssb).wait()

        @pl.when(s < TAIL)
        def _():
            off_e = base_e + (FULL * NT + s) * CH
            off_d = base_d + (FULL * NT + s) * CH
            pltpu.sync_copy(src_hbm.at[pl.ds(off_e, CH)], isa)
            pltpu.sync_copy(dst_hbm.at[pl.ds(off_d, CH)], ida)
            pltpu.async_copy(h_hbm.at[isa], ra, sem).wait()
            pltpu.sync_copy(ra, acc.at[ida], add=True)

        plsc.subcore_barrier()

        @pl.when(s < NT - 1)
        def _():
            pltpu.sync_copy(
                acc.at[pl.ds(r0, OWN), :],
                sum_hbm.at[pl.ds(c * N + r0, OWN), pl.ds(p * 128, 128)])

        @pl.when(s == NT - 1)
        def _():
            pltpu.sync_copy(
                acc.at[pl.ds(r0, LAST), :],
                sum_hbm.at[pl.ds(c * N + r0, LAST), pl.ds(p * 128, 128)])

        plsc.subcore_barrier()


@functools.lru_cache(maxsize=None)
def _layer2_call():
    return pl.kernel(
        _sc_layer2,
        out_type=[jax.ShapeDtypeStruct((NC * N, D_H), jnp.float32)],
        mesh=_mesh(),
        scratch_types=[
            pltpu.VMEM_SHARED((N, 128), jnp.float32),
            pltpu.VMEM((CH,), jnp.int32),
            pltpu.VMEM((CH,), jnp.int32),
            pltpu.VMEM((CH,), jnp.int32),
            pltpu.VMEM((CH,), jnp.int32),
            pltpu.VMEM((CH, 128), jnp.float32),
            pltpu.VMEM((CH, 128), jnp.float32),
            pltpu.SemaphoreType.DMA,
            pltpu.SemaphoreType.DMA,
            pltpu.SemaphoreType.DMA,
            pltpu.SemaphoreType.DMA,
            pltpu.SemaphoreType.DMA,
            pltpu.SemaphoreType.DMA,
            pltpu.SemaphoreType.DMA,
        ],
    )


def _tc_dense1(sum_ref, cnt_ref, xr_ref, wrel_ref, wroot_ref, b_ref, out_ref):
    cnt = jnp.maximum(cnt_ref[:, 0:1], 1.0)
    agg = sum_ref[...] / cnt
    h = jnp.dot(agg, wrel_ref[0], preferred_element_type=jnp.float32)
    h = h + jnp.dot(xr_ref[...], wroot_ref[0], preferred_element_type=jnp.float32)
    h = h + b_ref[0]
    h = jnp.maximum(h, 0.0)
    out_ref[0] = h[:, :128]
    out_ref[1] = h[:, 128:]


_dense1_call = pl.pallas_call(
    _tc_dense1,
    grid=(2, N // BR),
    in_specs=[
        pl.BlockSpec((BR, D_IN), lambda t, r: (t * (N // BR) + r, 0)),
        pl.BlockSpec((BR, 128), lambda t, r: (t * (N // BR) + r, 0)),
        pl.BlockSpec((BR, D_IN), lambda t, r: (t * (N // BR) + r, 0)),
        pl.BlockSpec((1, D_IN, D_H), lambda t, r: (t, 0, 0)),
        pl.BlockSpec((1, D_IN, D_H), lambda t, r: (t, 0, 0)),
        pl.BlockSpec((1, 1, D_H), lambda t, r: (t, 0, 0)),
    ],
    out_specs=pl.BlockSpec((2, BR, 128), lambda t, r: (1 - t, r, 0)),
    out_shape=jax.ShapeDtypeStruct((4, N, 128), jnp.float32),
)


def _tc_dense2(sum_ref, cnt_ref, h4_ref, wrel_ref, wroot_ref, b_ref,
               lw_ref, lb_ref, out_ref):
    cnt = jnp.maximum(cnt_ref[:, 0:1], 1.0)
    agg = sum_ref[...] / cnt
    o = jnp.dot(agg, wrel_ref[0], preferred_element_type=jnp.float32)
    o = o + jnp.dot(h4_ref[0], wroot_ref[0, :128, :],
                    preferred_element_type=jnp.float32)
    o = o + jnp.dot(h4_ref[1], wroot_ref[0, 128:, :],
                    preferred_element_type=jnp.float32)
    o = o + b_ref[0]
    out_ref[0] = jnp.dot(o, lw_ref[0], preferred_element_type=jnp.float32) \
        + lb_ref[0]


_dense2_call = pl.pallas_call(
    _tc_dense2,
    grid=(2, N // BR),
    in_specs=[
        pl.BlockSpec((BR, D_H), lambda t, r: (t * (N // BR) + r, 0)),
        pl.BlockSpec((BR, 128), lambda t, r: (t * (N // BR) + r, 0)),
        pl.BlockSpec((2, BR, 128), lambda t, r: (1 - t, r, 0)),
        pl.BlockSpec((1, D_H, D_OUT), lambda t, r: (t, 0, 0)),
        pl.BlockSpec((1, D_H, D_OUT), lambda t, r: (t, 0, 0)),
        pl.BlockSpec((1, 1, D_OUT), lambda t, r: (t, 0, 0)),
        pl.BlockSpec((1, D_OUT, D_OUT), lambda t, r: (t, 0, 0)),
        pl.BlockSpec((1, 1, D_OUT), lambda t, r: (t, 0, 0)),
    ],
    out_specs=pl.BlockSpec((1, BR, D_OUT), lambda t, r: (t, r, 0)),
    out_shape=jax.ShapeDtypeStruct((2, N, D_OUT), jnp.float32),
)


def kernel(x_user, x_item, edge_index_user_rates_item, edge_index_item_rated_by_user,
           l1_ui_Wrel, l1_ui_Wroot, l1_ui_b, l1_iu_Wrel, l1_iu_Wroot, l1_iu_b,
           l2_ui_Wrel, l2_ui_Wroot, l2_ui_b, l2_iu_Wrel, l2_iu_Wroot, l2_iu_b,
           lin_user_W, lin_user_b, lin_item_W, lin_item_b):
    src_ui = edge_index_user_rates_item[0].astype(jnp.int32)
    dst_ui = edge_index_user_rates_item[1].astype(jnp.int32)
    src_iu = edge_index_item_rated_by_user[0].astype(jnp.int32)
    dst_iu = edge_index_item_rated_by_user[1].astype(jnp.int32)

    x_all = jnp.concatenate([x_user, x_item], axis=0)
    src1 = jnp.concatenate([src_ui, src_iu + N])
    dst1 = jnp.concatenate([dst_ui, dst_iu])
    src2 = jnp.concatenate([src_ui, src_ui + N,
                            src_iu + 2 * N, src_iu + 3 * N])
    ones128 = jnp.ones((CH, 128), jnp.float32)
    zeros = jnp.zeros((N, D_IN), jnp.float32)

    sum1, cnt = _layer1_call()(x_all, src1, dst1, ones128, zeros)

    xr = jnp.concatenate([x_item, x_user], axis=0)
    w1rel = jnp.stack([l1_ui_Wrel, l1_iu_Wrel])
    w1root = jnp.stack([l1_ui_Wroot, l1_iu_Wroot])
    b1 = jnp.stack([l1_ui_b, l1_iu_b])[:, None, :]
    h4 = _dense1_call(sum1, cnt, xr, w1rel, w1root, b1)

    h_tab = h4.reshape(4 * N, 128)
    (sum2,) = _layer2_call()(h_tab, src2, dst1, zeros)

    w2rel = jnp.stack([l2_ui_Wrel, l2_iu_Wrel])
    w2root = jnp.stack([l2_ui_Wroot, l2_iu_Wroot])
    b2 = jnp.stack([l2_ui_b, l2_iu_b])[:, None, :]
    lw = jnp.stack([lin_item_W, lin_user_W])
    lb = jnp.stack([lin_item_b, lin_user_b])[:, None, :]
    out = _dense2_call(sum2, cnt, h4, w2rel, w2root, b2, lw, lb)
    return (out[1], out[0])


# speedup vs baseline: 6.0187x; 1.0016x over previous
import functools

import jax
import jax.numpy as jnp
from jax import lax
from jax.experimental import pallas as pl
from jax.experimental.pallas import tpu as pltpu
from jax.experimental.pallas import tpu_sc as plsc

N = 10000
D_IN = 128
D_H = 256
D_OUT = 128
E = 320000
NT = 16
NC = 2
CH = 128
CHUNKS = E // CH
FULL = CHUNKS // NT
HFULL = FULL // 2
TAIL = CHUNKS - FULL * NT
OWN = 640
LAST = N - OWN * (NT - 1)
BR = 1000


@functools.lru_cache(maxsize=None)
def _mesh():
    return plsc.VectorSubcoreMesh(core_axis_name="c", subcore_axis_name="s",
                                  num_cores=NC, num_subcores=NT)


def _sc_layer1(x_hbm, src_hbm, dst_hbm, ones_hbm, zeros_hbm,
               sum_hbm, cnt_hbm,
               acc, isa, ida, isb, idb, ra, rb, ones_v,
               sga, sgb, ssa, ssb, sia, sib, sem):
    c = lax.axis_index("c")
    s = lax.axis_index("s")
    r0 = s * OWN
    base_e = c * E

    def eoff(k):
        return base_e + (s + k * NT) * CH

    pltpu.sync_copy(ones_hbm, ones_v)

    for p in range(2):
        @pl.when(s < NT - 1)
        def _():
            pltpu.sync_copy(zeros_hbm.at[pl.ds(r0, OWN), :],
                            acc.at[pl.ds(r0, OWN), :])

        @pl.when(s == NT - 1)
        def _():
            pltpu.sync_copy(zeros_hbm.at[pl.ds(r0, LAST), :],
                            acc.at[pl.ds(r0, LAST), :])

        plsc.subcore_barrier()

        if p == 0:
            pltpu.sync_copy(src_hbm.at[pl.ds(eoff(0), CH)], isa)
            pltpu.sync_copy(dst_hbm.at[pl.ds(eoff(0), CH)], ida)
            pltpu.async_copy(x_hbm.at[isa], ra, sga)

            def body(j, carry):
                @pl.when(j > 0)
                def _():
                    pltpu.make_async_copy(rb, acc.at[idb], ssb).wait()

                pltpu.async_copy(src_hbm.at[pl.ds(eoff(2 * j + 1), CH)],
                                 isb, sib)
                pltpu.async_copy(dst_hbm.at[pl.ds(eoff(2 * j + 1), CH)],
                                 idb, sib)
                pltpu.make_async_copy(x_hbm.at[isa], ra, sga).wait()
                pltpu.make_async_copy(
                    src_hbm.at[pl.ds(eoff(0), CH)], isb, sib).wait()
                pltpu.make_async_copy(
                    dst_hbm.at[pl.ds(eoff(0), CH)], idb, sib).wait()
                pltpu.async_copy(x_hbm.at[isb], rb, sgb)
                pltpu.async_copy(ra, acc.at[ida], ssa, add=True)
                pltpu.make_async_copy(ra, acc.at[ida], ssa).wait()

                @pl.when(j < HFULL - 1)
                def _():
                    pltpu.async_copy(
                        src_hbm.at[pl.ds(eoff(2 * j + 2), CH)], isa, sia)
                    pltpu.async_copy(
                        dst_hbm.at[pl.ds(eoff(2 * j + 2), CH)], ida, sia)

                pltpu.make_async_copy(x_hbm.at[isb], rb, sgb).wait()
                pltpu.async_copy(rb, acc.at[idb], ssb, add=True)

                @pl.when(j < HFULL - 1)
                def _():
                    pltpu.make_async_copy(
                        src_hbm.at[pl.ds(eoff(0), CH)], isa, sia).wait()
                    pltpu.make_async_copy(
                        dst_hbm.at[pl.ds(eoff(0), CH)], ida, sia).wait()
                    pltpu.async_copy(x_hbm.at[isa], ra, sga)

                return carry

            lax.fori_loop(0, HFULL, body, 0)
            pltpu.make_async_copy(rb, acc.at[idb], ssb).wait()

            @pl.when(s < TAIL)
            def _():
                off = base_e + (FULL * NT + s) * CH
                pltpu.sync_copy(src_hbm.at[pl.ds(off, CH)], isa)
                pltpu.sync_copy(dst_hbm.at[pl.ds(off, CH)], ida)
                pltpu.async_copy(x_hbm.at[isa], ra, sem).wait()
                pltpu.sync_copy(ra, acc.at[ida], add=True)
        else:
            pltpu.sync_copy(dst_hbm.at[pl.ds(eoff(0), CH)], ida)

            def body(j, carry):
                pltpu.async_copy(ones_v, acc.at[ida], ssa, add=True)

                @pl.when(j > 0)
                def _():
                    pltpu.make_async_copy(ones_v, acc.at[idb], ssb).wait()

                pltpu.async_copy(dst_hbm.at[pl.ds(eoff(2 * j + 1), CH)],
                                 idb, sib)
                pltpu.make_async_copy(
                    dst_hbm.at[pl.ds(eoff(0), CH)], idb, sib).wait()
                pltpu.async_copy(ones_v, acc.at[idb], ssb, add=True)
                pltpu.make_async_copy(ones_v, acc.at[ida], ssa).wait()

                @pl.when(j < HFULL - 1)
                def _():
                    pltpu.async_copy(
                        dst_hbm.at[pl.ds(eoff(2 * j + 2), CH)], ida, sia)
                    pltpu.make_async_copy(
                        dst_hbm.at[pl.ds(eoff(0), CH)], ida, sia).wait()

                return carry

            lax.fori_loop(0, HFULL, body, 0)
            pltpu.make_async_copy(ones_v, acc.at[idb], ssb).wait()

            @pl.when(s < TAIL)
            def _():
                off = base_e + (FULL * NT + s) * CH
                pltpu.sync_copy(dst_hbm.at[pl.ds(off, CH)], ida)
                pltpu.sync_copy(ones_v, acc.at[ida], add=True)

        plsc.subcore_barrier()

        out_hbm = sum_hbm if p == 0 else cnt_hbm

        @pl.when(s < NT - 1)
        def _():
            pltpu.sync_copy(acc.at[pl.ds(r0, OWN), :],
                            out_hbm.at[pl.ds(c * N + r0, OWN), :])

        @pl.when(s == NT - 1)
        def _():
            pltpu.sync_copy(acc.at[pl.ds(r0, LAST), :],
                            out_hbm.at[pl.ds(c * N + r0, LAST), :])

        if p == 0:
            plsc.subcore_barrier()


@functools.lru_cache(maxsize=None)
def _layer1_call():
    return pl.kernel(
        _sc_layer1,
        out_type=[jax.ShapeDtypeStruct((NC * N, D_IN), jnp.float32),
                  jax.ShapeDtypeStruct((NC * N, 128), jnp.float32)],
        mesh=_mesh(),
        scratch_types=[
            pltpu.VMEM_SHARED((N, D_IN), jnp.float32),
            pltpu.VMEM((CH,), jnp.int32),
            pltpu.VMEM((CH,), jnp.int32),
            pltpu.VMEM((CH,), jnp.int32),
            pltpu.VMEM((CH,), jnp.int32),
            pltpu.VMEM((CH, D_IN), jnp.float32),
            pltpu.VMEM((CH, D_IN), jnp.float32),
            pltpu.VMEM((CH, 128), jnp.float32),
            pltpu.SemaphoreType.DMA,
            pltpu.SemaphoreType.DMA,
            pltpu.SemaphoreType.DMA,
            pltpu.SemaphoreType.DMA,
            pltpu.SemaphoreType.DMA,
            pltpu.SemaphoreType.DMA,
            pltpu.SemaphoreType.DMA,
        ],
    )


def _sc_layer2(h_hbm, src_hbm, dst_hbm, zeros_hbm,
               sum_hbm,
               acc, isa, ida, isb, idb, ra, rb,
               sga, sgb, ssa, ssb, sia, sib, sem):
    c = lax.axis_index("c")
    s = lax.axis_index("s")
    r0 = s * OWN

    for p in range(2):
        base_e = (2 * c + p) * E
        base_d = c * E

        def eoff(k):
            return base_e + (s + k * NT) * CH

        def doff(k):
            return base_d + (s + k * NT) * CH

        @pl.when(s < NT - 1)
        def _():
            pltpu.sync_copy(zeros_hbm.at[pl.ds(r0, OWN), :],
                            acc.at[pl.ds(r0, OWN), :])

        @pl.when(s == NT - 1)
        def _():
            pltpu.sync_copy(zeros_hbm.at[pl.ds(r0, LAST), :],
                            acc.at[pl.ds(r0, LAST), :])

        plsc.subcore_barrier()

        pltpu.sync_copy(src_hbm.at[pl.ds(eoff(0), CH)], isa)
        pltpu.sync_copy(dst_hbm.at[pl.ds(doff(0), CH)], ida)
        pltpu.async_copy(h_hbm.at[isa], ra, sga)

        def body(j, carry):
            @pl.when(j > 0)
            def _():
                pltpu.make_async_copy(rb, acc.at[idb], ssb).wait()

            pltpu.async_copy(src_hbm.at[pl.ds(eoff(2 * j + 1), CH)],
                             isb, sib)
            pltpu.async_copy(dst_hbm.at[pl.ds(doff(2 * j + 1), CH)],
                             idb, sib)
            pltpu.make_async_copy(h_hbm.at[isa], ra, sga).wait()
            pltpu.make_async_copy(
                src_hbm.at[pl.ds(eoff(0), CH)], isb, sib).wait()
            pltpu.make_async_copy(
                dst_hbm.at[pl.ds(doff(0), CH)], idb, sib).wait()
            pltpu.async_copy(h_hbm.at[isb], rb, sgb)
            pltpu.async_copy(ra, acc.at[ida], ssa, add=True)
            pltpu.make_async_copy(ra, acc.at[ida], ssa).wait()

            @pl.when(j < HFULL - 1)
            def _():
                pltpu.async_copy(
                    src_hbm.at[pl.ds(eoff(2 * j + 2), CH)], isa, sia)
                pltpu.async_copy(
                    dst_hbm.at[pl.ds(doff(2 * j + 2), CH)], ida, sia)

            pltpu.make_async_copy(h_hbm.at[isb], rb, sgb).wait()
            pltpu.async_copy(rb, acc.at[idb], ssb, add=True)

            @pl.when(j < HFULL - 1)
            def _():
                pltpu.make_async_copy(
                    src_hbm.at[pl.ds(eoff(0), CH)], isa, sia).wait()
                pltpu.make_async_copy(
                    dst_hbm.at[pl.ds(doff(0), CH)], ida, sia).wait()
                pltpu.async_copy(h_hbm.at[isa], ra, sga)

            return carry

        lax.fori_loop(0, HFULL, body, 0)
        pltpu.make_async_copy(rb, acc.at[idb], ssb).wait()

        @pl.when(s < TAIL)
        def _():
            off_e = base_e + (FULL * NT + s) * CH
            off_d = base_d + (FULL * NT + s) * CH
            pltpu.sync_copy(src_hbm.at[pl.ds(off_e, CH)], isa)
            pltpu.sync_copy(dst_hbm.at[pl.ds(off_d, CH)], ida)
            pltpu.async_copy(h_hbm.at[isa], ra, sem).wait()
            pltpu.sync_copy(ra, acc.at[ida], add=True)

        plsc.subcore_barrier()

        @pl.when(s < NT - 1)
        def _():
            pltpu.sync_copy(
                acc.at[pl.ds(r0, OWN), :],
                sum_hbm.at[pl.ds(c * N + r0, OWN), pl.ds(p * 128, 128)])

        @pl.when(s == NT - 1)
        def _():
            pltpu.sync_copy(
                acc.at[pl.ds(r0, LAST), :],
                sum_hbm.at[pl.ds(c * N + r0, LAST), pl.ds(p * 128, 128)])

        plsc.subcore_barrier()


@functools.lru_cache(maxsize=None)
def _layer2_call():
    return pl.kernel(
        _sc_layer2,
        out_type=[jax.ShapeDtypeStruct((NC * N, D_H), jnp.float32)],
        mesh=_mesh(),
        scratch_types=[
            pltpu.VMEM_SHARED((N, 128), jnp.float32),
            pltpu.VMEM((CH,), jnp.int32),
            pltpu.VMEM((CH,), jnp.int32),
            pltpu.VMEM((CH,), jnp.int32),
            pltpu.VMEM((CH,), jnp.int32),
            pltpu.VMEM((CH, 128), jnp.float32),
            pltpu.VMEM((CH, 128), jnp.float32),
            pltpu.SemaphoreType.DMA,
            pltpu.SemaphoreType.DMA,
            pltpu.SemaphoreType.DMA,
            pltpu.SemaphoreType.DMA,
            pltpu.SemaphoreType.DMA,
            pltpu.SemaphoreType.DMA,
            pltpu.SemaphoreType.DMA,
        ],
    )


def _tc_dense1(sum_ref, cnt_ref, xr_ref, wrel_ref, wroot_ref, b_ref, out_ref):
    cnt = jnp.maximum(cnt_ref[:, 0:1], 1.0)
    agg = sum_ref[...] / cnt
    h = jnp.dot(agg, wrel_ref[0], preferred_element_type=jnp.float32)
    h = h + jnp.dot(xr_ref[...], wroot_ref[0], preferred_element_type=jnp.float32)
    h = h + b_ref[0]
    h = jnp.maximum(h, 0.0)
    out_ref[0] = h[:, :128]
    out_ref[1] = h[:, 128:]


_dense1_call = pl.pallas_call(
    _tc_dense1,
    grid=(2, N // BR),
    in_specs=[
        pl.BlockSpec((BR, D_IN), lambda t, r: (t * (N // BR) + r, 0)),
        pl.BlockSpec((BR, 128), lambda t, r: (t * (N // BR) + r, 0)),
        pl.BlockSpec((BR, D_IN), lambda t, r: (t * (N // BR) + r, 0)),
        pl.BlockSpec((1, D_IN, D_H), lambda t, r: (t, 0, 0)),
        pl.BlockSpec((1, D_IN, D_H), lambda t, r: (t, 0, 0)),
        pl.BlockSpec((1, 1, D_H), lambda t, r: (t, 0, 0)),
    ],
    out_specs=pl.BlockSpec((2, BR, 128), lambda t, r: (1 - t, r, 0)),
    out_shape=jax.ShapeDtypeStruct((4, N, 128), jnp.float32),
)


def _tc_dense2(sum_ref, cnt_ref, h4_ref, wrel_ref, wroot_ref, b_ref,
               lw_ref, lb_ref, out_ref):
    cnt = jnp.maximum(cnt_ref[:, 0:1], 1.0)
    agg = sum_ref[...] / cnt
    o = jnp.dot(agg, wrel_ref[0], preferred_element_type=jnp.float32)
    o = o + jnp.dot(h4_ref[0], wroot_ref[0, :128, :],
                    preferred_element_type=jnp.float32)
    o = o + jnp.dot(h4_ref[1], wroot_ref[0, 128:, :],
                    preferred_element_type=jnp.float32)
    o = o + b_ref[0]
    out_ref[0] = jnp.dot(o, lw_ref[0], preferred_element_type=jnp.float32) \
        + lb_ref[0]


_dense2_call = pl.pallas_call(
    _tc_dense2,
    grid=(2, N // BR),
    in_specs=[
        pl.BlockSpec((BR, D_H), lambda t, r: (t * (N // BR) + r, 0)),
        pl.BlockSpec((BR, 128), lambda t, r: (t * (N // BR) + r, 0)),
        pl.BlockSpec((2, BR, 128), lambda t, r: (1 - t, r, 0)),
        pl.BlockSpec((1, D_H, D_OUT), lambda t, r: (t, 0, 0)),
        pl.BlockSpec((1, D_H, D_OUT), lambda t, r: (t, 0, 0)),
        pl.BlockSpec((1, 1, D_OUT), lambda t, r: (t, 0, 0)),
        pl.BlockSpec((1, D_OUT, D_OUT), lambda t, r: (t, 0, 0)),
        pl.BlockSpec((1, 1, D_OUT), lambda t, r: (t, 0, 0)),
    ],
    out_specs=pl.BlockSpec((1, BR, D_OUT), lambda t, r: (t, r, 0)),
    out_shape=jax.ShapeDtypeStruct((2, N, D_OUT), jnp.float32),
)


def kernel(x_user, x_item, edge_index_user_rates_item, edge_index_item_rated_by_user,
           l1_ui_Wrel, l1_ui_Wroot, l1_ui_b, l1_iu_Wrel, l1_iu_Wroot, l1_iu_b,
           l2_ui_Wrel, l2_ui_Wroot, l2_ui_b, l2_iu_Wrel, l2_iu_Wroot, l2_iu_b,
           lin_user_W, lin_user_b, lin_item_W, lin_item_b):
    src_ui = edge_index_user_rates_item[0].astype(jnp.int32)
    dst_ui = edge_index_user_rates_item[1].astype(jnp.int32)
    src_iu = edge_index_item_rated_by_user[0].astype(jnp.int32)
    dst_iu = edge_index_item_rated_by_user[1].astype(jnp.int32)

    x_all = jnp.concatenate([x_user, x_item], axis=0)
    src1 = jnp.concatenate([src_ui, src_iu + N])
    dst1 = jnp.concatenate([dst_ui, dst_iu])
    src2 = jnp.concatenate([src_ui, src_ui + N,
                            src_iu + 2 * N, src_iu + 3 * N])
    ones128 = jnp.ones((CH, 128), jnp.float32)
    zeros = jnp.zeros((N, D_IN), jnp.float32)

    sum1, cnt = _layer1_call()(x_all, src1, dst1, ones128, zeros)

    xr = jnp.concatenate([x_item, x_user], axis=0)
    w1rel = jnp.stack([l1_ui_Wrel, l1_iu_Wrel])
    w1root = jnp.stack([l1_ui_Wroot, l1_iu_Wroot])
    b1 = jnp.stack([l1_ui_b, l1_iu_b])[:, None, :]
    h4 = _dense1_call(sum1, cnt, xr, w1rel, w1root, b1)

    h_tab = h4.reshape(4 * N, 128)
    (sum2,) = _layer2_call()(h_tab, src2, dst1, zeros)

    w2rel = jnp.stack([l2_ui_Wrel, l2_iu_Wrel])
    w2root = jnp.stack([l2_ui_Wroot, l2_iu_Wroot])
    b2 = jnp.stack([l2_ui_b, l2_iu_b])[:, None, :]
    lw = jnp.stack([lin_item_W, lin_user_W])
    lb = jnp.stack([lin_item_b, lin_user_b])[:, None, :]
    out = _dense2_call(sum2, cnt, h4, w2rel, w2root, b2, lw, lb)
    return (out[1], out[0])
